# baseline (speedup 1.0000x reference)
import functools

import jax
import jax.numpy as jnp
from jax import lax
from jax.experimental import pallas as pl
from jax.experimental.pallas import tpu as pltpu
from jax.experimental.pallas import tpu_sc as plsc

NUM_EMB = 1000000
DIM = 32
BATCH = 4096
HIST = 200

NC = 2
NS = 16
NW = NC * NS
BB = BATCH // NW

NB = 5

_mesh = plsc.VectorSubcoreMesh(core_axis_name="c", subcore_axis_name="s")


@functools.partial(
    pl.kernel,
    mesh=_mesh,
    out_type=jax.ShapeDtypeStruct((HIST, DIM // 8, BATCH // BB, 8, BB), jnp.float32),
    scratch_types=[
        pltpu.VMEM((HIST, BB), jnp.int32),
        pltpu.VMEM((NB, BB, DIM), jnp.float32),
        pltpu.VMEM((NB, DIM // 8, 8, BB), jnp.float32),
        [pltpu.SemaphoreType.DMA] * NB,
        [pltpu.SemaphoreType.DMA] * NB,
    ],
    compiler_params=pltpu.CompilerParams(
        use_tc_tiling_on_sc=False, needs_layout_passes=False
    ),
)
def _sc_gather(idxt_hbm, table_hbm, out_hbm, idx_v, rows_v, tp_v, gsems, osems):
    wid = lax.axis_index("s") * NC + lax.axis_index("c")
    b0 = wid * BB

    lanes = lax.broadcasted_iota(jnp.int32, (16,), 0)
    row_ids = [lanes + 16 * k for k in range(BB // 16)]

    def fire_gather(g, b):
        pltpu.async_copy(
            table_hbm.at[idx_v.at[g]], rows_v.at[b], gsems[b]
        )

    def wait_gather(b):
        pltpu.make_async_copy(
            table_hbm.at[pl.ds(0, BB)], rows_v.at[b], gsems[b]
        ).wait()

    def fire_write(g, b):
        pltpu.async_copy(tp_v.at[b], out_hbm.at[g, :, wid], osems[b])

    def wait_write(b):
        pltpu.make_async_copy(
            table_hbm.at[pl.ds(0, BB)], rows_v.at[b], osems[b]
        ).wait()

    def transpose(b):
        def ch_body(ch, carry):
            r_hi = ch >> 3
            r_lo = ch & 7
            colv = jnp.full((16,), 0, jnp.int32) + ch
            for k in range(BB // 16):
                vals = plsc.load_gather(rows_v.at[b], [row_ids[k], colv])
                tp_v[b, r_hi, r_lo, pl.ds(16 * k, 16)] = vals
            return carry

        lax.fori_loop(0, DIM, ch_body, 0)

    pltpu.sync_copy(idxt_hbm.at[:, pl.ds(b0, BB)], idx_v)

    for b in range(NB - 1):
        fire_gather(b, b)

    def outer(p, carry):
        for b in range(NB):
            g = p * NB + b
            bf = (b + NB - 1) % NB
            gf = g + NB - 1
            if b == 0:
                @pl.when(p >= 1)
                def _():
                    wait_write(bf)

                fire_gather(gf, bf)
            else:
                wait_write(bf)

                @pl.when(gf < HIST)
                def _():
                    fire_gather(gf, bf)

            wait_gather(b)
            transpose(b)
            fire_write(g, b)
        return carry

    lax.fori_loop(0, HIST // NB, outer, 0)
    wait_write((HIST - 1) % NB)


def kernel(inp, table):
    x = _sc_gather(inp.T, table)
    y = jnp.transpose(x, (2, 4, 0, 1, 3))
    return y.reshape(BATCH, HIST, DIM)


# speedup vs baseline: 1.1168x; 1.1168x over previous
import functools

import jax
import jax.numpy as jnp
from jax import lax
from jax.experimental import pallas as pl
from jax.experimental.pallas import tpu as pltpu
from jax.experimental.pallas import tpu_sc as plsc

NUM_EMB = 1000000
DIM = 32
BATCH = 4096
HIST = 200

NC = 2
NS = 16
NW = NC * NS
BB = BATCH // NW

NB = 5

N_TCOL = NUM_EMB // 128
V_TAIL = NUM_EMB - N_TCOL * 128
RING = 3

_mesh = plsc.VectorSubcoreMesh(core_axis_name="c", subcore_axis_name="s")


@functools.partial(
    pl.kernel,
    mesh=_mesh,
    out_type=jax.ShapeDtypeStruct((NUM_EMB // 4, 128), jnp.float32),
    scratch_types=[
        pltpu.VMEM((RING, DIM, 128), jnp.float32),
        pltpu.VMEM((RING, DIM, 128), jnp.float32),
        [pltpu.SemaphoreType.DMA] * RING,
        [pltpu.SemaphoreType.DMA] * RING,
    ],
    compiler_params=pltpu.CompilerParams(
        use_tc_tiling_on_sc=True, needs_layout_passes=False
    ),
)
def _sc_relayout(tt_hbm, tail_hbm, out_hbm, src_v, dst_v, rsems, wsems):
    wid = lax.axis_index("s") * NC + lax.axis_index("c")
    lanes = lax.broadcasted_iota(jnp.int32, (16,), 0)
    row_lo = lanes
    row_hi = lanes + 16

    def fire_read(c, b):
        pltpu.async_copy(
            tt_hbm.at[:, pl.ds(c * 128, 128)], src_v.at[b], rsems[b]
        )

    def wait_read(b):
        pltpu.make_async_copy(
            tt_hbm.at[:, pl.ds(0, 128)], src_v.at[b], rsems[b]
        ).wait()

    def fire_write(c, b):
        pltpu.async_copy(
            dst_v.at[b], out_hbm.at[pl.ds(c * 32, 32)], wsems[b]
        )

    def wait_write(b):
        pltpu.make_async_copy(
            tt_hbm.at[:, pl.ds(0, 128)], src_v.at[b], wsems[b]
        ).wait()

    def transpose(b, n_rows):
        def j_body(j, carry):
            cols = [jnp.full((16,), 0, jnp.int32) + (4 * j + a) for a in range(4)]
            vals = [
                plsc.load_gather(
                    src_v.at[b], [row_lo if h % 2 == 0 else row_hi, cols[h // 2]]
                )
                for h in range(8)
            ]
            for h in range(8):
                dst_v[b, j, pl.ds(16 * h, 16)] = vals[h]
            return carry

        lax.fori_loop(0, n_rows, j_body, 0)

    for b in range(RING):
        fire_read(wid + b * NW, b)

    n_outer = (N_TCOL // NW + RING) // RING

    def outer(r, carry):
        for b in range(RING):
            rnd = r * RING + b
            c = rnd * NW + wid

            @pl.when(c < N_TCOL)
            def _():
                wait_read(b)

                @pl.when(rnd >= RING)
                def _():
                    wait_write(b)

                transpose(b, DIM)
                fire_write(c, b)
                cn = c + RING * NW

                @pl.when(cn < N_TCOL)
                def _():
                    fire_read(cn, b)
        return carry

    lax.fori_loop(0, n_outer, outer, 0)

    for b in range(RING):
        wait_write(b)

    @pl.when(wid == 0)
    def _():
        pltpu.sync_copy(tail_hbm, dst_v.at[0, pl.ds(0, V_TAIL // 4)])
        pltpu.sync_copy(
            dst_v.at[0, pl.ds(0, V_TAIL // 4)],
            out_hbm.at[pl.ds(N_TCOL * 32, V_TAIL // 4)],
        )


@functools.partial(
    pl.kernel,
    mesh=_mesh,
    out_type=jax.ShapeDtypeStruct((HIST, DIM // 8, BATCH // BB, 8, BB), jnp.float32),
    scratch_types=[
        pltpu.VMEM((HIST, BB), jnp.int32),
        pltpu.VMEM((NB, BB, DIM), jnp.float32),
        pltpu.VMEM((NB, DIM // 8, 8, BB), jnp.float32),
        [pltpu.SemaphoreType.DMA] * NB,
        [pltpu.SemaphoreType.DMA] * NB,
    ],
    compiler_params=pltpu.CompilerParams(
        use_tc_tiling_on_sc=False, needs_layout_passes=False
    ),
)
def _sc_gather(idxt_hbm, table_hbm, out_hbm, idx_v, rows_v, tp_v, gsems, osems):
    wid = lax.axis_index("s") * NC + lax.axis_index("c")
    b0 = wid * BB

    lanes = lax.broadcasted_iota(jnp.int32, (16,), 0)
    row_ids = [lanes + 16 * k for k in range(BB // 16)]

    def fire_gather(g, b):
        pltpu.async_copy(
            table_hbm.at[idx_v.at[g]], rows_v.at[b], gsems[b]
        )

    def wait_gather(b):
        pltpu.make_async_copy(
            table_hbm.at[pl.ds(0, BB)], rows_v.at[b], gsems[b]
        ).wait()

    def fire_write(g, b):
        pltpu.async_copy(tp_v.at[b], out_hbm.at[g, :, wid], osems[b])

    def wait_write(b):
        pltpu.make_async_copy(
            table_hbm.at[pl.ds(0, BB)], rows_v.at[b], osems[b]
        ).wait()

    def transpose(b):
        def ch_body(i, carry):
            ch0 = 2 * i
            ch1 = 2 * i + 1
            col0 = jnp.full((16,), 0, jnp.int32) + ch0
            col1 = col0 + 1
            vals0 = [
                plsc.load_gather(rows_v.at[b], [row_ids[k], col0])
                for k in range(BB // 16)
            ]
            vals1 = [
                plsc.load_gather(rows_v.at[b], [row_ids[k], col1])
                for k in range(BB // 16)
            ]
            for k in range(BB // 16):
                tp_v[b, ch0 >> 3, ch0 & 7, pl.ds(16 * k, 16)] = vals0[k]
            for k in range(BB // 16):
                tp_v[b, ch1 >> 3, ch1 & 7, pl.ds(16 * k, 16)] = vals1[k]
            return carry

        lax.fori_loop(0, DIM // 2, ch_body, 0)

    pltpu.sync_copy(idxt_hbm.at[:, pl.ds(b0, BB)], idx_v)

    for b in range(NB - 1):
        fire_gather(b, b)

    def outer(p, carry):
        for b in range(NB):
            g = p * NB + b
            bf = (b + NB - 1) % NB
            gf = g + NB - 1
            if b == 0:
                @pl.when(p >= 1)
                def _():
                    wait_write(bf)

                fire_gather(gf, bf)
            else:
                wait_write(bf)

                @pl.when(gf < HIST)
                def _():
                    fire_gather(gf, bf)

            wait_gather(b)
            transpose(b)
            fire_write(g, b)
        return carry

    lax.fori_loop(0, HIST // NB, outer, 0)
    wait_write((HIST - 1) % NB)


def kernel(inp, table):
    tail = table[N_TCOL * 128:].reshape(V_TAIL // 4, 128)
    trm = _sc_relayout(table.T, tail).reshape(NUM_EMB, DIM)
    x = _sc_gather(inp.T, trm)
    y = jnp.transpose(x, (2, 4, 0, 1, 3))
    return y.reshape(BATCH, HIST, DIM)


# speedup vs baseline: 1.1999x; 1.0745x over previous
import functools

import jax
import jax.numpy as jnp
from jax import lax
from jax.experimental import pallas as pl
from jax.experimental.pallas import tpu as pltpu
from jax.experimental.pallas import tpu_sc as plsc

NUM_EMB = 1000000
DIM = 32
BATCH = 4096
HIST = 200

NC = 2
NS = 16
NW = NC * NS
BB = BATCH // NW

NB = 5

N_TCOL = NUM_EMB // 128
V_TAIL = NUM_EMB - N_TCOL * 128
RING = 3

_mesh = plsc.VectorSubcoreMesh(core_axis_name="c", subcore_axis_name="s")


@functools.partial(
    pl.kernel,
    mesh=_mesh,
    out_type=jax.ShapeDtypeStruct((NUM_EMB // 4, 128), jnp.float32),
    scratch_types=[
        pltpu.VMEM((RING, DIM, 128), jnp.float32),
        pltpu.VMEM((RING, DIM, 128), jnp.float32),
        [pltpu.SemaphoreType.DMA] * RING,
        [pltpu.SemaphoreType.DMA] * RING,
    ],
    compiler_params=pltpu.CompilerParams(
        use_tc_tiling_on_sc=True, needs_layout_passes=False
    ),
)
def _sc_relayout(tt_hbm, tail_hbm, out_hbm, src_v, dst_v, rsems, wsems):
    wid = lax.axis_index("s") * NC + lax.axis_index("c")
    lanes = lax.broadcasted_iota(jnp.int32, (16,), 0)
    row_lo = lanes
    row_hi = lanes + 16

    def fire_read(c, b):
        pltpu.async_copy(
            tt_hbm.at[:, pl.ds(c * 128, 128)], src_v.at[b], rsems[b]
        )

    def wait_read(b):
        pltpu.make_async_copy(
            tt_hbm.at[:, pl.ds(0, 128)], src_v.at[b], rsems[b]
        ).wait()

    def fire_write(c, b):
        pltpu.async_copy(
            dst_v.at[b], out_hbm.at[pl.ds(c * 32, 32)], wsems[b]
        )

    def wait_write(b):
        pltpu.make_async_copy(
            tt_hbm.at[:, pl.ds(0, 128)], src_v.at[b], wsems[b]
        ).wait()

    def transpose(b, n_rows):
        @plsc.parallel_loop(0, n_rows, unroll=2)
        def j_body(j):
            cols = [jnp.full((16,), 0, jnp.int32) + (4 * j + a) for a in range(4)]
            vals = [
                plsc.load_gather(
                    src_v.at[b], [row_lo if h % 2 == 0 else row_hi, cols[h // 2]]
                )
                for h in range(8)
            ]
            for h in range(8):
                dst_v[b, j, pl.ds(16 * h, 16)] = vals[h]

    for b in range(RING):
        fire_read(wid + b * NW, b)

    n_outer = (N_TCOL // NW + RING) // RING

    def outer(r, carry):
        for b in range(RING):
            rnd = r * RING + b
            c = rnd * NW + wid

            @pl.when(c < N_TCOL)
            def _():
                wait_read(b)

                @pl.when(rnd >= RING)
                def _():
                    wait_write(b)

                transpose(b, DIM)
                fire_write(c, b)
                cn = c + RING * NW

                @pl.when(cn < N_TCOL)
                def _():
                    fire_read(cn, b)
        return carry

    lax.fori_loop(0, n_outer, outer, 0)

    for b in range(RING):
        wait_write(b)

    @pl.when(wid == 0)
    def _():
        pltpu.sync_copy(tail_hbm, dst_v.at[0, pl.ds(0, V_TAIL // 4)])
        pltpu.sync_copy(
            dst_v.at[0, pl.ds(0, V_TAIL // 4)],
            out_hbm.at[pl.ds(N_TCOL * 32, V_TAIL // 4)],
        )


@functools.partial(
    pl.kernel,
    mesh=_mesh,
    out_type=jax.ShapeDtypeStruct((HIST, DIM // 8, BATCH // BB, 8, BB), jnp.float32),
    scratch_types=[
        pltpu.VMEM((HIST, BB), jnp.int32),
        pltpu.VMEM((NB, BB, DIM), jnp.float32),
        pltpu.VMEM((NB, DIM // 8, 8, BB), jnp.float32),
        [pltpu.SemaphoreType.DMA] * NB,
        [pltpu.SemaphoreType.DMA] * NB,
    ],
    compiler_params=pltpu.CompilerParams(
        use_tc_tiling_on_sc=False, needs_layout_passes=False
    ),
)
def _sc_gather(idxt_hbm, table_hbm, out_hbm, idx_v, rows_v, tp_v, gsems, osems):
    wid = lax.axis_index("s") * NC + lax.axis_index("c")
    b0 = wid * BB

    lanes = lax.broadcasted_iota(jnp.int32, (16,), 0)
    row_ids = [lanes + 16 * k for k in range(BB // 16)]

    def fire_gather(g, b):
        pltpu.async_copy(
            table_hbm.at[idx_v.at[g]], rows_v.at[b], gsems[b]
        )

    def wait_gather(b):
        pltpu.make_async_copy(
            table_hbm.at[pl.ds(0, BB)], rows_v.at[b], gsems[b]
        ).wait()

    def fire_write(g, b):
        pltpu.async_copy(tp_v.at[b], out_hbm.at[g, :, wid], osems[b])

    def wait_write(b):
        pltpu.make_async_copy(
            table_hbm.at[pl.ds(0, BB)], rows_v.at[b], osems[b]
        ).wait()

    def transpose(b):
        @plsc.parallel_loop(0, DIM // 2, unroll=2)
        def ch_body(i):
            ch0 = 2 * i
            ch1 = 2 * i + 1
            col0 = jnp.full((16,), 0, jnp.int32) + ch0
            col1 = col0 + 1
            vals0 = [
                plsc.load_gather(rows_v.at[b], [row_ids[k], col0])
                for k in range(BB // 16)
            ]
            vals1 = [
                plsc.load_gather(rows_v.at[b], [row_ids[k], col1])
                for k in range(BB // 16)
            ]
            for k in range(BB // 16):
                tp_v[b, ch0 >> 3, ch0 & 7, pl.ds(16 * k, 16)] = vals0[k]
            for k in range(BB // 16):
                tp_v[b, ch1 >> 3, ch1 & 7, pl.ds(16 * k, 16)] = vals1[k]

    pltpu.sync_copy(idxt_hbm.at[:, pl.ds(b0, BB)], idx_v)

    for b in range(NB - 1):
        fire_gather(b, b)

    def outer(p, carry):
        for b in range(NB):
            g = p * NB + b
            bf = (b + NB - 1) % NB
            gf = g + NB - 1
            if b == 0:
                @pl.when(p >= 1)
                def _():
                    wait_write(bf)

                fire_gather(gf, bf)
            else:
                wait_write(bf)

                @pl.when(gf < HIST)
                def _():
                    fire_gather(gf, bf)

            wait_gather(b)
            transpose(b)
            fire_write(g, b)
        return carry

    lax.fori_loop(0, HIST // NB, outer, 0)
    wait_write((HIST - 1) % NB)


def kernel(inp, table):
    tail = table[N_TCOL * 128:].reshape(V_TAIL // 4, 128)
    trm = _sc_relayout(table.T, tail).reshape(NUM_EMB, DIM)
    x = _sc_gather(inp.T, trm)
    y = jnp.transpose(x, (2, 4, 0, 1, 3))
    return y.reshape(BATCH, HIST, DIM)


# speedup vs baseline: 1.2480x; 1.0401x over previous
import functools

import jax
import jax.numpy as jnp
from jax import lax
from jax.experimental import pallas as pl
from jax.experimental.pallas import tpu as pltpu
from jax.experimental.pallas import tpu_sc as plsc

NUM_EMB = 1000000
DIM = 32
BATCH = 4096
HIST = 200

NC = 2
NS = 16
NW = NC * NS
BB = BATCH // NW

NB = 5

N_TCOL = NUM_EMB // 128
V_TAIL = NUM_EMB - N_TCOL * 128
RING = 3

_mesh = plsc.VectorSubcoreMesh(core_axis_name="c", subcore_axis_name="s")


@functools.partial(
    pl.kernel,
    mesh=_mesh,
    out_type=jax.ShapeDtypeStruct((NUM_EMB // 4, 128), jnp.float32),
    scratch_types=[
        pltpu.VMEM((RING, DIM, 128), jnp.float32),
        pltpu.VMEM((RING, DIM, 128), jnp.float32),
        [pltpu.SemaphoreType.DMA] * RING,
        [pltpu.SemaphoreType.DMA] * RING,
    ],
    compiler_params=pltpu.CompilerParams(
        use_tc_tiling_on_sc=True, needs_layout_passes=False
    ),
)
def _sc_relayout(tt_hbm, tail_hbm, out_hbm, src_v, dst_v, rsems, wsems):
    wid = lax.axis_index("s") * NC + lax.axis_index("c")
    lanes = lax.broadcasted_iota(jnp.int32, (16,), 0)
    row_lo = lanes
    row_hi = lanes + 16

    def fire_read(c, b):
        pltpu.async_copy(
            tt_hbm.at[:, pl.ds(c * 128, 128)], src_v.at[b], rsems[b]
        )

    def wait_read(b):
        pltpu.make_async_copy(
            tt_hbm.at[:, pl.ds(0, 128)], src_v.at[b], rsems[b]
        ).wait()

    def fire_write(c, b):
        pltpu.async_copy(
            dst_v.at[b], out_hbm.at[pl.ds(c * 32, 32)], wsems[b]
        )

    def wait_write(b):
        pltpu.make_async_copy(
            tt_hbm.at[:, pl.ds(0, 128)], src_v.at[b], wsems[b]
        ).wait()

    def transpose(b, n_rows):
        @plsc.parallel_loop(0, n_rows, unroll=4)
        def j_body(j):
            cols = [jnp.full((16,), 0, jnp.int32) + (4 * j + a) for a in range(4)]
            vals = [
                plsc.load_gather(
                    src_v.at[b], [row_lo if h % 2 == 0 else row_hi, cols[h // 2]]
                )
                for h in range(8)
            ]
            for h in range(8):
                dst_v[b, j, pl.ds(16 * h, 16)] = vals[h]

    for b in range(RING):
        fire_read(wid + b * NW, b)

    n_outer = (N_TCOL // NW + RING) // RING

    def outer(r, carry):
        for b in range(RING):
            rnd = r * RING + b
            c = rnd * NW + wid

            @pl.when(c < N_TCOL)
            def _():
                wait_read(b)

                @pl.when(rnd >= RING)
                def _():
                    wait_write(b)

                transpose(b, DIM)
                fire_write(c, b)
                cn = c + RING * NW

                @pl.when(cn < N_TCOL)
                def _():
                    fire_read(cn, b)
        return carry

    lax.fori_loop(0, n_outer, outer, 0)

    for b in range(RING):
        wait_write(b)

    @pl.when(wid == 0)
    def _():
        pltpu.sync_copy(tail_hbm, dst_v.at[0, pl.ds(0, V_TAIL // 4)])
        pltpu.sync_copy(
            dst_v.at[0, pl.ds(0, V_TAIL // 4)],
            out_hbm.at[pl.ds(N_TCOL * 32, V_TAIL // 4)],
        )


@functools.partial(
    pl.kernel,
    mesh=_mesh,
    out_type=jax.ShapeDtypeStruct((HIST, DIM // 8, BATCH // BB, 8, BB), jnp.float32),
    scratch_types=[
        pltpu.VMEM((HIST, BB), jnp.int32),
        pltpu.VMEM((NB, BB, DIM), jnp.float32),
        pltpu.VMEM((NB, DIM // 8, 8, BB), jnp.float32),
        [pltpu.SemaphoreType.DMA] * NB,
        [pltpu.SemaphoreType.DMA] * NB,
    ],
    compiler_params=pltpu.CompilerParams(
        use_tc_tiling_on_sc=False, needs_layout_passes=False
    ),
)
def _sc_gather(idxt_hbm, table_hbm, out_hbm, idx_v, rows_v, tp_v, gsems, osems):
    wid = lax.axis_index("s") * NC + lax.axis_index("c")
    b0 = wid * BB

    lanes = lax.broadcasted_iota(jnp.int32, (16,), 0)
    row_ids = [lanes + 16 * k for k in range(BB // 16)]

    def fire_gather(g, b):
        pltpu.async_copy(
            table_hbm.at[idx_v.at[g]], rows_v.at[b], gsems[b]
        )

    def wait_gather(b):
        pltpu.make_async_copy(
            table_hbm.at[pl.ds(0, BB)], rows_v.at[b], gsems[b]
        ).wait()

    def fire_write(g, b):
        pltpu.async_copy(tp_v.at[b], out_hbm.at[g, :, wid], osems[b])

    def wait_write(b):
        pltpu.make_async_copy(
            table_hbm.at[pl.ds(0, BB)], rows_v.at[b], osems[b]
        ).wait()

    def transpose(b):
        @plsc.parallel_loop(0, DIM // 2, unroll=4)
        def ch_body(i):
            ch0 = 2 * i
            ch1 = 2 * i + 1
            col0 = jnp.full((16,), 0, jnp.int32) + ch0
            col1 = col0 + 1
            vals0 = [
                plsc.load_gather(rows_v.at[b], [row_ids[k], col0])
                for k in range(BB // 16)
            ]
            vals1 = [
                plsc.load_gather(rows_v.at[b], [row_ids[k], col1])
                for k in range(BB // 16)
            ]
            for k in range(BB // 16):
                tp_v[b, ch0 >> 3, ch0 & 7, pl.ds(16 * k, 16)] = vals0[k]
            for k in range(BB // 16):
                tp_v[b, ch1 >> 3, ch1 & 7, pl.ds(16 * k, 16)] = vals1[k]

    pltpu.sync_copy(idxt_hbm.at[:, pl.ds(b0, BB)], idx_v)

    for b in range(NB - 1):
        fire_gather(b, b)

    def outer(p, carry):
        for b in range(NB):
            g = p * NB + b
            bf = (b + NB - 1) % NB
            gf = g + NB - 1
            if b == 0:
                @pl.when(p >= 1)
                def _():
                    wait_write(bf)

                fire_gather(gf, bf)
            else:
                wait_write(bf)

                @pl.when(gf < HIST)
                def _():
                    fire_gather(gf, bf)

            wait_gather(b)
            transpose(b)
            fire_write(g, b)
        return carry

    lax.fori_loop(0, HIST // NB, outer, 0)
    wait_write((HIST - 1) % NB)


def kernel(inp, table):
    tail = table[N_TCOL * 128:].reshape(V_TAIL // 4, 128)
    trm = _sc_relayout(table.T, tail).reshape(NUM_EMB, DIM)
    x = _sc_gather(inp.T, trm)
    y = jnp.transpose(x, (2, 4, 0, 1, 3))
    return y.reshape(BATCH, HIST, DIM)
